# Initial kernel scaffold; baseline (speedup 1.0000x reference)
import jax
import jax.numpy as jnp
from jax.experimental import pallas as pl


def kernel(xyz, points_fea, W0, b0, W1, b1, W2, b2):
    raise NotImplementedError("write your pallas kernel here")


# speedup vs baseline: 6.3779x; 6.3779x over previous
import functools

import jax
import jax.numpy as jnp
from jax import lax
from jax.experimental import pallas as pl
from jax.experimental.pallas import tpu as pltpu
from jax.experimental.pallas import tpu_sc as plsc

B, N, D = 16, 4096, 64
S, K = 1024, 32
CP = 8


TS_TOPK = 256


def _topk_body(q_ref, pt_ref, idx_ref):
    b = pl.program_id(0)
    q = q_ref[0]
    pt = pt_ref[0]
    pn2 = jnp.sum(pt * pt, axis=0, keepdims=True)
    qn2 = jnp.sum(q * q, axis=1, keepdims=True)
    dot = jnp.dot(q, pt, preferred_element_type=jnp.float32)
    d2 = jnp.maximum(pn2 + qn2 - 2.0 * dot, 0.0)
    bits0 = lax.bitcast_convert_type(d2, jnp.int32)
    iota = lax.broadcasted_iota(jnp.int32, (TS_TOPK, N), 1)
    kiota = lax.broadcasted_iota(jnp.int32, (TS_TOPK, K), 1)
    bigi = jnp.int32(0x7FFFFFFF)

    def body(k, carry):
        bits, out = carry
        mval = jnp.min(bits, axis=1, keepdims=True)
        sel = jnp.where(bits == mval, iota, jnp.int32(N))
        idxk = jnp.min(sel, axis=1, keepdims=True)
        bits = jnp.where(iota == idxk, bigi, bits)
        out = jnp.where(kiota == k, idxk, out)
        return bits, out

    _, out = lax.fori_loop(
        0, K, body, (bits0, jnp.zeros((TS_TOPK, K), jnp.int32))
    )
    idx_ref[0] = out + b * N


def _topk_call(xyz_pad, xyzT_pad):
    return pl.pallas_call(
        _topk_body,
        grid=(B, S // TS_TOPK),
        in_specs=[
            pl.BlockSpec((1, TS_TOPK, CP), lambda b, i: (b, i, 0)),
            pl.BlockSpec((1, CP, N), lambda b, i: (b, 0, 0)),
        ],
        out_specs=pl.BlockSpec((1, TS_TOPK, K), lambda b, i: (b, i, 0)),
        out_shape=jax.ShapeDtypeStruct((B, S, K), jnp.int32),
    )(xyz_pad, xyzT_pad)


def _pretransform_body(xyz_ref, fea_ref, w0x_ref, w0f_ref, g_ref, c_ref):
    xyzp = xyz_ref[0]
    fea = fea_ref[0]
    w0x = w0x_ref[...]
    w0f = w0f_ref[...]
    g = jnp.dot(xyzp, w0x, preferred_element_type=jnp.float32)
    g = g + jnp.dot(fea, w0f, preferred_element_type=jnp.float32)
    g_ref[0] = g
    c_ref[0] = jnp.dot(xyzp[:S], w0x, preferred_element_type=jnp.float32)


def _pretransform_call(xyz_pad, points_fea, w0x, w0f):
    c1 = w0x.shape[1]
    return pl.pallas_call(
        _pretransform_body,
        grid=(B,),
        in_specs=[
            pl.BlockSpec((1, N, CP), lambda b: (b, 0, 0)),
            pl.BlockSpec((1, N, D), lambda b: (b, 0, 0)),
            pl.BlockSpec((CP, c1), lambda b: (0, 0)),
            pl.BlockSpec((D, c1), lambda b: (0, 0)),
        ],
        out_specs=[
            pl.BlockSpec((1, N, c1), lambda b: (b, 0, 0)),
            pl.BlockSpec((1, S, c1), lambda b: (b, 0, 0)),
        ],
        out_shape=[
            jax.ShapeDtypeStruct((B, N, c1), jnp.float32),
            jax.ShapeDtypeStruct((B, S, c1), jnp.float32),
        ],
    )(xyz_pad, points_fea, w0x, w0f)


NC, NS = 2, 16
NW = NC * NS
TOTAL = B * S * K
RW = TOTAL // NW
CH = 128


def _gather_body(table_hbm, idx_hbm, out_hbm, idx_v, rows_v, sem):
    wid = lax.axis_index("s") * NC + lax.axis_index("c")
    base = wid * RW

    def body(i, _):
        off = base + i * CH
        pltpu.sync_copy(idx_hbm.at[pl.ds(off, CH)], idx_v)
        pltpu.async_copy(table_hbm.at[idx_v], rows_v, sem).wait()
        pltpu.sync_copy(rows_v, out_hbm.at[pl.ds(off, CH)])
        return 0

    lax.fori_loop(0, RW // CH, body, 0)


def _gather_call(table, idx_flat):
    c1 = table.shape[1]
    mesh = plsc.VectorSubcoreMesh(
        core_axis_name="c", subcore_axis_name="s", num_cores=NC, num_subcores=NS
    )
    f = functools.partial(
        pl.kernel,
        out_type=jax.ShapeDtypeStruct((TOTAL, c1), jnp.float32),
        mesh=mesh,
        scratch_types=[
            pltpu.VMEM((CH,), jnp.int32),
            pltpu.VMEM((CH, c1), jnp.float32),
            pltpu.SemaphoreType.DMA,
        ],
    )(_gather_body)
    return f(table, idx_flat)


TS_MLP = 128
RMLP = TS_MLP * K


def _mlp_body(gg_ref, c_ref, b0_ref, w1_ref, b1_ref, w2_ref, b2_ref, out_ref):
    gg = gg_ref[0]
    cc = b0_ref[...] - c_ref[0]
    h = gg.reshape(TS_MLP, K, 128) + cc[:, None, :]
    h = jnp.maximum(h, 0.0).reshape(RMLP, 128)
    h = jnp.dot(h, w1_ref[...], preferred_element_type=jnp.float32)
    h = jnp.maximum(h + b1_ref[...], 0.0)
    h = jnp.dot(h, w2_ref[...], preferred_element_type=jnp.float32)
    h = jnp.maximum(h + b2_ref[...], 0.0)
    out_ref[0] = jnp.max(h.reshape(TS_MLP, K, 256), axis=1)


def _mlp_call(gg, cmat, b0, w1, b1, w2, b2):
    c1, c2, c3 = w1.shape[0], w2.shape[0], w2.shape[1]
    return pl.pallas_call(
        _mlp_body,
        grid=(B, S // TS_MLP),
        in_specs=[
            pl.BlockSpec((1, RMLP, c1), lambda b, i: (b, i, 0)),
            pl.BlockSpec((1, TS_MLP, c1), lambda b, i: (b, i, 0)),
            pl.BlockSpec((1, c1), lambda b, i: (0, 0)),
            pl.BlockSpec((c1, c2), lambda b, i: (0, 0)),
            pl.BlockSpec((1, c2), lambda b, i: (0, 0)),
            pl.BlockSpec((c2, c3), lambda b, i: (0, 0)),
            pl.BlockSpec((1, c3), lambda b, i: (0, 0)),
        ],
        out_specs=pl.BlockSpec((1, TS_MLP, c3), lambda b, i: (b, i, 0)),
        out_shape=jax.ShapeDtypeStruct((B, S, c3), jnp.float32),
    )(gg, cmat, b0, w1, b1, w2, b2)


def kernel(xyz, points_fea, W0, b0, W1, b1, W2, b2):
    sampled_xyz = xyz[:, :S, :]
    xyz_pad = jnp.pad(xyz, ((0, 0), (0, 0), (0, CP - 3)))
    xyzT_pad = jnp.transpose(xyz_pad, (0, 2, 1))
    w0x = jnp.pad(W0[:3], ((0, CP - 3), (0, 0)))
    w0f = W0[3:]

    idx = _topk_call(xyz_pad, xyzT_pad)
    g, cmat = _pretransform_call(xyz_pad, points_fea, w0x, w0f)
    gg = _gather_call(g.reshape(B * N, -1), idx.reshape(TOTAL))
    gg = gg.reshape(B, S * K, -1)
    out_fea = _mlp_call(
        gg, cmat, b0.reshape(1, -1), W1, b1.reshape(1, -1), W2, b2.reshape(1, -1)
    )
    return (sampled_xyz, out_fea)
